# Initial kernel scaffold; baseline (speedup 1.0000x reference)
import jax
import jax.numpy as jnp
from jax.experimental import pallas as pl


def kernel(h_x, t_x, edge_index, enrichment_weights, W_src, W_dst, att_src, att_dst, bias, enrichment_scale):
    raise NotImplementedError("write your pallas kernel here")


# speedup vs baseline: 77.5570x; 77.5570x over previous
import functools

import jax
import jax.numpy as jnp
from jax import lax
from jax.experimental import pallas as pl
from jax.experimental.pallas import tpu as pltpu
from jax.experimental.pallas import tpu_sc as plsc

N = 10000
D = 128
E = 320000
H = 2
C = 32
HC = H * C

NC = 2
NS = 16
NW = NC * NS
N_PAD = 10240
EB = 128
KB = 79
TE = KB * EB
E_PAD = NW * TE
DEN_ROWS = N_PAD * H // 128

_MESH = plsc.VectorSubcoreMesh(core_axis_name="c", subcore_axis_name="s")
_SC_PARAMS = pltpu.CompilerParams(use_tc_tiling_on_sc=False,
                                  needs_layout_passes=False)


def _prologue_body(hx_ref, tx_ref, wcat_ref, bh_ref, bt_ref, ew_ref, es_ref,
                   xs1_ref, xs2_ref, acat_ref, fac_ref):
    def _elu(x):
        return jnp.where(x > 0.0, x, jnp.exp(jnp.minimum(x, 0.0)) - 1.0)
    h_in = _elu(hx_ref[...])
    t_in = _elu(tx_ref[...])
    wcat = wcat_ref[...]
    h_comb = jnp.dot(h_in, wcat, preferred_element_type=jnp.float32)
    t_comb = jnp.dot(t_in, wcat, preferred_element_type=jnp.float32)
    xs1_ref[...] = h_comb[:, :HC]
    xs2_ref[...] = t_comb[:, :HC]
    acat_ref[...] = (
        jnp.dot(h_comb, bh_ref[...], preferred_element_type=jnp.float32)
        + jnp.dot(t_comb, bt_ref[...], preferred_element_type=jnp.float32))
    w = jnp.clip(ew_ref[...], 0.3, 3.0)
    m = jnp.max(w)
    e = jnp.exp(w - m)
    weighted = jnp.sum(w * e) / jnp.sum(e)
    sf = 0.5 * jnp.tanh(es_ref[0, 0])
    fac_ref[...] = jnp.broadcast_to(1.0 + sf * (weighted - 1.0), (1, 1))


@functools.partial(
    pl.kernel,
    out_type=[
        jax.ShapeDtypeStruct((NC, DEN_ROWS, 128), jnp.float32),
        jax.ShapeDtypeStruct((NC, DEN_ROWS, 128), jnp.float32),
        jax.ShapeDtypeStruct((NW, H, TE), jnp.float32),
        jax.ShapeDtypeStruct((NW, H, TE), jnp.float32),
    ],
    mesh=_MESH,
    compiler_params=_SC_PARAMS,
    scratch_types=[
        pltpu.VMEM((KB, EB), jnp.int32),
        pltpu.VMEM((KB, EB), jnp.int32),
        pltpu.VMEM((N_PAD, 4), jnp.float32),
        pltpu.VMEM((DEN_ROWS, 128), jnp.float32),
        pltpu.VMEM((H, EB), jnp.float32),
        pltpu.VMEM((2, 80), jnp.int32),
        pltpu.VMEM_SHARED((DEN_ROWS, 128), jnp.float32),
        pltpu.VMEM_SHARED((DEN_ROWS, 128), jnp.float32),
    ],
)
def _edge_logits(src_hbm, dst_hbm, a1_hbm, a2_hbm,
                 den1_hbm, den2_hbm, ex1_hbm, ex2_hbm,
                 src_loc, dst_loc, a_tab, den_loc, ex_buf, iota_idx,
                 spden1, spden2):
    c = lax.axis_index("c")
    s = lax.axis_index("s")
    wid = s * NC + c
    zeros16 = jnp.zeros((16,), jnp.float32)
    lane = lax.iota(jnp.int32, 16)

    for r in range(2):
        for q in range(5):
            iota_idx[r, pl.ds(q * 16, 16)] = lane + (r * 80 + q * 16)

    def zden(r, carry):
        for q in range(8):
            den_loc[r, pl.ds(q * 16, 16)] = zeros16
        return carry
    lax.fori_loop(0, DEN_ROWS, zden, 0)

    @pl.when(s < 10)
    def _zero_spden():
        pltpu.sync_copy(den_loc.at[pl.ds(0, 16)], spden1.at[pl.ds(s * 16, 16)])
        pltpu.sync_copy(den_loc.at[pl.ds(0, 16)], spden2.at[pl.ds(s * 16, 16)])

    pltpu.sync_copy(src_hbm.at[wid], src_loc)
    pltpu.sync_copy(dst_hbm.at[wid], dst_loc)
    plsc.subcore_barrier()

    def run_conv(a_hbm, reverse, first, spden, ex_hbm, den_hbm):
        pltpu.sync_copy(a_hbm, a_tab)

        if not first:
            def zden2(r, carry):
                for q in range(8):
                    den_loc[r, pl.ds(q * 16, 16)] = zeros16
                return carry
            lax.fori_loop(0, DEN_ROWS, zden2, 0)

        col_s0 = jnp.zeros((16,), jnp.int32)
        col_s1 = col_s0 + 1
        col_d0 = col_s0 + 2
        col_d1 = col_s0 + 3

        def blk(j, carry):
            for k in range(8):
                srcv = src_loc[j, pl.ds(k * 16, 16)]
                dstv = dst_loc[j, pl.ds(k * 16, 16)]
                gs, gd = (dstv, srcv) if reverse else (srcv, dstv)
                for h in range(H):
                    a_s = plsc.load_gather(a_tab, [gs, col_s0 if h == 0 else col_s1])
                    a_d = plsc.load_gather(a_tab, [gd, col_d0 if h == 0 else col_d1])
                    al = a_s + a_d
                    al = jnp.where(al >= 0.0, al, 0.2 * al)
                    exv = jnp.exp(al)
                    ex_buf[h, pl.ds(k * 16, 16)] = exv
                    f = gd * 2 + h
                    plsc.addupdate_scatter(
                        den_loc,
                        [lax.shift_right_logical(f, 7), lax.bitwise_and(f, 127)],
                        exv)
            pltpu.sync_copy(ex_buf, ex_hbm.at[wid, :, pl.ds(j * EB, EB)])
            return carry
        lax.fori_loop(0, KB, blk, 0)

        for half in range(2):
            pltpu.sync_copy(den_loc.at[pl.ds(half * 80, 80)],
                            spden.at[iota_idx.at[half]], add=True)

    run_conv(a1_hbm, False, True, spden1, ex1_hbm, den1_hbm)
    run_conv(a2_hbm, True, False, spden2, ex2_hbm, den2_hbm)

    plsc.subcore_barrier()

    @pl.when(s < 10)
    def _write_spden():
        pltpu.sync_copy(spden1.at[pl.ds(s * 16, 16)],
                        den1_hbm.at[c, pl.ds(s * 16, 16)])
        pltpu.sync_copy(spden2.at[pl.ds(s * 16, 16)],
                        den2_hbm.at[c, pl.ds(s * 16, 16)])


@functools.partial(
    pl.kernel,
    out_type=jax.ShapeDtypeStruct((NC, N_PAD, HC), jnp.float32),
    mesh=_MESH,
    compiler_params=_SC_PARAMS,
    scratch_types=[
        pltpu.VMEM((KB, EB), jnp.int32),
        pltpu.VMEM((KB, EB), jnp.int32),
        pltpu.VMEM((H, TE), jnp.float32),
        pltpu.VMEM((EB, HC), jnp.float32),
        pltpu.VMEM((EB, HC), jnp.float32),
        pltpu.VMEM_SHARED((N_PAD, HC), jnp.float32),
        pltpu.SemaphoreType.DMA,
    ],
)
def _messages(xs_hbm, gsrc_hbm, gdst_hbm, ex_hbm, out_hbm,
              gsrc, gdst, ex_loc, rbuf, zbuf, spout, sem):
    c = lax.axis_index("c")
    s = lax.axis_index("s")
    wid = s * NC + c
    zeros16 = jnp.zeros((16,), jnp.float32)

    def zrow(r, carry):
        for q in range(4):
            zbuf[r, pl.ds(q * 16, 16)] = zeros16
        return carry
    lax.fori_loop(0, EB, zrow, 0)
    for z in range(5):
        pltpu.sync_copy(zbuf, spout.at[pl.ds(s * 640 + z * EB, EB)])
    pltpu.sync_copy(gsrc_hbm.at[wid], gsrc)
    pltpu.sync_copy(gdst_hbm.at[wid], gdst)
    pltpu.sync_copy(ex_hbm.at[wid], ex_loc)
    plsc.subcore_barrier()

    zi16 = jnp.zeros((16,), jnp.int32)

    def blk(j, carry):
        pltpu.async_copy(xs_hbm.at[gsrc.at[j]], rbuf, sem).wait()
        base = j * EB
        for e in range(EB):
            col = zi16 + (base + e)
            for h in range(H):
                exs = plsc.load_gather(ex_loc, [zi16 + h, col])
                for t in range(2):
                    q = h * 2 + t
                    v = rbuf[e, pl.ds(q * 16, 16)]
                    rbuf[e, pl.ds(q * 16, 16)] = v * exs
        pltpu.sync_copy(rbuf, spout.at[gdst.at[j]], add=True)
        return carry
    lax.fori_loop(0, KB, blk, 0)

    plsc.subcore_barrier()
    pltpu.sync_copy(spout.at[pl.ds(s * 640, 640)],
                    out_hbm.at[c, pl.ds(s * 640, 640)])


def _epilogue_body(o1a_ref, o1b_ref, o2a_ref, o2b_ref, d1_ref, d2_ref,
                   b_ref, f_ref, t_ref, h_ref):
    fac = f_ref[0, 0]
    bias = b_ref[...]
    t_ref[...] = ((o1a_ref[...] + o1b_ref[...]) / d1_ref[...] + bias) * fac
    h_ref[...] = ((o2a_ref[...] + o2b_ref[...]) / d2_ref[...] + bias) * fac


def kernel(h_x, t_x, edge_index, enrichment_weights, W_src, W_dst,
           att_src, att_dst, bias, enrichment_scale):
    f32 = jnp.float32
    W_cat = jnp.concatenate([W_src, W_dst], axis=1)

    a_srcv = att_src.reshape(H, C)
    a_dstv = att_dst.reshape(H, C)
    z32 = jnp.zeros((C,), f32)
    zc = jnp.zeros((HC,), f32)
    zc2 = jnp.zeros((D,), f32)
    colA0 = jnp.concatenate([a_srcv[0], z32])
    colA1 = jnp.concatenate([z32, a_srcv[1]])
    colD0 = jnp.concatenate([a_dstv[0], z32])
    colD1 = jnp.concatenate([z32, a_dstv[1]])
    Bh = jnp.stack([
        jnp.concatenate([colA0, zc]), jnp.concatenate([colA1, zc]),
        zc2, zc2, zc2, zc2,
        jnp.concatenate([zc, colD0]), jnp.concatenate([zc, colD1])], axis=1)
    Bt = jnp.stack([
        zc2, zc2,
        jnp.concatenate([zc, colD0]), jnp.concatenate([zc, colD1]),
        jnp.concatenate([colA0, zc]), jnp.concatenate([colA1, zc]),
        zc2, zc2], axis=1)

    ew_r = enrichment_weights.reshape(E // 128, 128)
    es2 = enrichment_scale.reshape(1, 1)

    xs1, xs2, a_cat, fac = pl.pallas_call(
        _prologue_body,
        out_shape=[
            jax.ShapeDtypeStruct((N, HC), f32),
            jax.ShapeDtypeStruct((N, HC), f32),
            jax.ShapeDtypeStruct((N, 8), f32),
            jax.ShapeDtypeStruct((1, 1), f32),
        ],
    )(h_x, t_x, W_cat, Bh, Bt, ew_r, es2)

    sent = jnp.array([[-1e30, -1e30, 0.0, 0.0]], f32)
    padz = jnp.zeros((N_PAD - N - 1, 4), f32)
    a1 = jnp.concatenate([a_cat[:, 0:4], sent, padz], axis=0)
    a2 = jnp.concatenate([a_cat[:, 4:8], sent, padz], axis=0)
    xs_padz = jnp.zeros((N_PAD - N, HC), f32)
    xs1p = jnp.concatenate([xs1, xs_padz], axis=0)
    xs2p = jnp.concatenate([xs2, xs_padz], axis=0)

    padi = jnp.full((E_PAD - E,), N, jnp.int32)
    src_p = jnp.concatenate([edge_index[0], padi]).reshape(NW, KB, EB)
    dst_p = jnp.concatenate([edge_index[1], padi]).reshape(NW, KB, EB)

    den1_p, den2_p, ex1, ex2 = _edge_logits(src_p, dst_p, a1, a2)
    o1 = _messages(xs1p, src_p, dst_p, ex1)
    o2 = _messages(xs2p, dst_p, src_p, ex2)

    den1 = (den1_p[0] + den1_p[1]).reshape(N_PAD, H)[:N]
    den2 = (den2_p[0] + den2_p[1]).reshape(N_PAD, H)[:N]
    d1 = jnp.repeat(den1, C, axis=1) + 1e-16
    d2 = jnp.repeat(den2, C, axis=1) + 1e-16

    t_rep, h_rep = pl.pallas_call(
        _epilogue_body,
        out_shape=[
            jax.ShapeDtypeStruct((N, HC), f32),
            jax.ShapeDtypeStruct((N, HC), f32),
        ],
    )(o1[0, :N], o1[1, :N], o2[0, :N], o2[1, :N], d1, d2,
      bias.reshape(1, HC), fac)
    return (h_rep, t_rep)


# speedup vs baseline: 111.5097x; 1.4378x over previous
import functools

import jax
import jax.numpy as jnp
from jax import lax
from jax.experimental import pallas as pl
from jax.experimental.pallas import tpu as pltpu
from jax.experimental.pallas import tpu_sc as plsc

N = 10000
D = 128
E = 320000
H = 2
C = 32
HC = H * C

NC = 2
NS = 16
NW = NC * NS
N_PAD = 10240
EB = 128
KB = 79
TE = KB * EB
E_PAD = NW * TE
DEN_ROWS = N_PAD * H // 128

_MESH = plsc.VectorSubcoreMesh(core_axis_name="c", subcore_axis_name="s")
_SC_PARAMS = pltpu.CompilerParams(use_tc_tiling_on_sc=False,
                                  needs_layout_passes=False)


def _prologue_body(hx_ref, tx_ref, wcat_ref, bh_ref, bt_ref, ew_ref, es_ref,
                   xs1_ref, xs2_ref, acat_ref, fac_ref):
    def _elu(x):
        return jnp.where(x > 0.0, x, jnp.exp(jnp.minimum(x, 0.0)) - 1.0)
    h_in = _elu(hx_ref[...])
    t_in = _elu(tx_ref[...])
    wcat = wcat_ref[...]
    h_comb = jnp.dot(h_in, wcat, preferred_element_type=jnp.float32)
    t_comb = jnp.dot(t_in, wcat, preferred_element_type=jnp.float32)
    xs1_ref[...] = h_comb[:, :HC]
    xs2_ref[...] = t_comb[:, :HC]
    acat_ref[...] = (
        jnp.dot(h_comb, bh_ref[...], preferred_element_type=jnp.float32)
        + jnp.dot(t_comb, bt_ref[...], preferred_element_type=jnp.float32))
    w = jnp.clip(ew_ref[...], 0.3, 3.0)
    m = jnp.max(w)
    e = jnp.exp(w - m)
    weighted = jnp.sum(w * e) / jnp.sum(e)
    sf = 0.5 * jnp.tanh(es_ref[0, 0])
    fac_ref[...] = jnp.broadcast_to(1.0 + sf * (weighted - 1.0), (1, 1))


@functools.partial(
    pl.kernel,
    out_type=[
        jax.ShapeDtypeStruct((NC, DEN_ROWS, 128), jnp.float32),
        jax.ShapeDtypeStruct((NC, DEN_ROWS, 128), jnp.float32),
        jax.ShapeDtypeStruct((NW, H, TE), jnp.float32),
        jax.ShapeDtypeStruct((NW, H, TE), jnp.float32),
    ],
    mesh=_MESH,
    compiler_params=_SC_PARAMS,
    scratch_types=[
        pltpu.VMEM((KB, EB), jnp.int32),
        pltpu.VMEM((KB, EB), jnp.int32),
        pltpu.VMEM((N_PAD, 4), jnp.float32),
        pltpu.VMEM((DEN_ROWS, 128), jnp.float32),
        pltpu.VMEM((H, EB), jnp.float32),
        pltpu.VMEM((2, 80), jnp.int32),
        pltpu.VMEM_SHARED((DEN_ROWS, 128), jnp.float32),
        pltpu.VMEM_SHARED((DEN_ROWS, 128), jnp.float32),
    ],
)
def _edge_logits(src_hbm, dst_hbm, a1_hbm, a2_hbm,
                 den1_hbm, den2_hbm, ex1_hbm, ex2_hbm,
                 src_loc, dst_loc, a_tab, den_loc, ex_buf, iota_idx,
                 spden1, spden2):
    c = lax.axis_index("c")
    s = lax.axis_index("s")
    wid = s * NC + c
    zeros16 = jnp.zeros((16,), jnp.float32)
    lane = lax.iota(jnp.int32, 16)

    for r in range(2):
        for q in range(5):
            iota_idx[r, pl.ds(q * 16, 16)] = lane + (r * 80 + q * 16)

    def zden(r, carry):
        for q in range(8):
            den_loc[r, pl.ds(q * 16, 16)] = zeros16
        return carry
    lax.fori_loop(0, DEN_ROWS, zden, 0)

    @pl.when(s < 10)
    def _zero_spden():
        pltpu.sync_copy(den_loc.at[pl.ds(0, 16)], spden1.at[pl.ds(s * 16, 16)])
        pltpu.sync_copy(den_loc.at[pl.ds(0, 16)], spden2.at[pl.ds(s * 16, 16)])

    pltpu.sync_copy(src_hbm.at[wid], src_loc)
    pltpu.sync_copy(dst_hbm.at[wid], dst_loc)
    plsc.subcore_barrier()

    def run_conv(a_hbm, reverse, first, spden, ex_hbm, den_hbm):
        pltpu.sync_copy(a_hbm, a_tab)

        if not first:
            def zden2(r, carry):
                for q in range(8):
                    den_loc[r, pl.ds(q * 16, 16)] = zeros16
                return carry
            lax.fori_loop(0, DEN_ROWS, zden2, 0)

        col_s0 = jnp.zeros((16,), jnp.int32)
        col_s1 = col_s0 + 1
        col_d0 = col_s0 + 2
        col_d1 = col_s0 + 3

        def blk(j, carry):
            for k in range(8):
                srcv = src_loc[j, pl.ds(k * 16, 16)]
                dstv = dst_loc[j, pl.ds(k * 16, 16)]
                gs, gd = (dstv, srcv) if reverse else (srcv, dstv)
                for h in range(H):
                    a_s = plsc.load_gather(a_tab, [gs, col_s0 if h == 0 else col_s1])
                    a_d = plsc.load_gather(a_tab, [gd, col_d0 if h == 0 else col_d1])
                    al = a_s + a_d
                    al = jnp.where(al >= 0.0, al, 0.2 * al)
                    exv = jnp.exp(al)
                    ex_buf[h, pl.ds(k * 16, 16)] = exv
                    f = gd * 2 + h
                    plsc.addupdate_scatter(
                        den_loc,
                        [lax.shift_right_logical(f, 7), lax.bitwise_and(f, 127)],
                        exv)
            pltpu.sync_copy(ex_buf, ex_hbm.at[wid, :, pl.ds(j * EB, EB)])
            return carry
        lax.fori_loop(0, KB, blk, 0)

        for half in range(2):
            pltpu.sync_copy(den_loc.at[pl.ds(half * 80, 80)],
                            spden.at[iota_idx.at[half]], add=True)

    run_conv(a1_hbm, False, True, spden1, ex1_hbm, den1_hbm)
    run_conv(a2_hbm, True, False, spden2, ex2_hbm, den2_hbm)

    plsc.subcore_barrier()

    @pl.when(s < 10)
    def _write_spden():
        pltpu.sync_copy(spden1.at[pl.ds(s * 16, 16)],
                        den1_hbm.at[c, pl.ds(s * 16, 16)])
        pltpu.sync_copy(spden2.at[pl.ds(s * 16, 16)],
                        den2_hbm.at[c, pl.ds(s * 16, 16)])


@functools.partial(
    pl.kernel,
    out_type=jax.ShapeDtypeStruct((NC, N_PAD, HC), jnp.float32),
    mesh=_MESH,
    compiler_params=_SC_PARAMS,
    scratch_types=[
        pltpu.VMEM((KB, EB), jnp.int32),
        pltpu.VMEM((KB, EB), jnp.int32),
        pltpu.VMEM((H, TE), jnp.float32),
        pltpu.VMEM((EB, HC), jnp.float32),
        pltpu.VMEM((EB, HC), jnp.float32),
        pltpu.VMEM((EB, HC), jnp.float32),
        pltpu.VMEM_SHARED((N_PAD, HC), jnp.float32),
        pltpu.SemaphoreType.DMA,
        pltpu.SemaphoreType.DMA,
    ],
)
def _messages(xs_hbm, gsrc_hbm, gdst_hbm, ex_hbm, out_hbm,
              gsrc, gdst, ex_loc, rbuf0, rbuf1, sbuf, spout, sem0, sem1):
    c = lax.axis_index("c")
    s = lax.axis_index("s")
    wid = s * NC + c
    zeros16 = jnp.zeros((16,), jnp.float32)

    def zrow(r, carry):
        for q in range(4):
            sbuf[r, pl.ds(q * 16, 16)] = zeros16
        return carry
    lax.fori_loop(0, EB, zrow, 0)
    for z in range(5):
        pltpu.sync_copy(sbuf, spout.at[pl.ds(s * 640 + z * EB, EB)])
    pltpu.sync_copy(gsrc_hbm.at[wid], gsrc)
    pltpu.sync_copy(gdst_hbm.at[wid], gdst)
    pltpu.sync_copy(ex_hbm.at[wid], ex_loc)
    plsc.subcore_barrier()

    zi16 = jnp.zeros((16,), jnp.int32)

    def scale(rbuf, j):
        base = j * EB
        for e in range(EB):
            col = zi16 + (base + e)
            ex0 = plsc.load_gather(ex_loc, [zi16, col])
            ex1 = plsc.load_gather(ex_loc, [zi16 + 1, col])
            for q in range(4):
                m = ex0 if q < 2 else ex1
                sbuf[e, pl.ds(q * 16, 16)] = rbuf[e, pl.ds(q * 16, 16)] * m

    pltpu.async_copy(xs_hbm.at[gsrc.at[0]], rbuf0, sem0)

    def pair(jj, carry):
        j0 = jj * 2
        j1 = j0 + 1
        pltpu.async_copy(xs_hbm.at[gsrc.at[j1]], rbuf1, sem1)
        pltpu.make_async_copy(xs_hbm.at[gsrc.at[j0]], rbuf0, sem0).wait()
        scale(rbuf0, j0)
        pltpu.async_copy(xs_hbm.at[gsrc.at[j0 + 2]], rbuf0, sem0)
        pltpu.sync_copy(sbuf, spout.at[gdst.at[j0]], add=True)
        pltpu.make_async_copy(xs_hbm.at[gsrc.at[j1]], rbuf1, sem1).wait()
        scale(rbuf1, j1)
        pltpu.sync_copy(sbuf, spout.at[gdst.at[j1]], add=True)
        return carry
    lax.fori_loop(0, KB // 2, pair, 0)

    jt = KB - 1
    pltpu.make_async_copy(xs_hbm.at[gsrc.at[jt]], rbuf0, sem0).wait()
    scale(rbuf0, jt)
    pltpu.sync_copy(sbuf, spout.at[gdst.at[jt]], add=True)

    plsc.subcore_barrier()
    pltpu.sync_copy(spout.at[pl.ds(s * 640, 640)],
                    out_hbm.at[c, pl.ds(s * 640, 640)])


def _epilogue_body(o1a_ref, o1b_ref, o2a_ref, o2b_ref, d1_ref, d2_ref,
                   b_ref, f_ref, t_ref, h_ref):
    fac = f_ref[0, 0]
    bias = b_ref[...]
    t_ref[...] = ((o1a_ref[...] + o1b_ref[...]) / d1_ref[...] + bias) * fac
    h_ref[...] = ((o2a_ref[...] + o2b_ref[...]) / d2_ref[...] + bias) * fac


def kernel(h_x, t_x, edge_index, enrichment_weights, W_src, W_dst,
           att_src, att_dst, bias, enrichment_scale):
    f32 = jnp.float32
    W_cat = jnp.concatenate([W_src, W_dst], axis=1)

    a_srcv = att_src.reshape(H, C)
    a_dstv = att_dst.reshape(H, C)
    z32 = jnp.zeros((C,), f32)
    zc = jnp.zeros((HC,), f32)
    zc2 = jnp.zeros((D,), f32)
    colA0 = jnp.concatenate([a_srcv[0], z32])
    colA1 = jnp.concatenate([z32, a_srcv[1]])
    colD0 = jnp.concatenate([a_dstv[0], z32])
    colD1 = jnp.concatenate([z32, a_dstv[1]])
    Bh = jnp.stack([
        jnp.concatenate([colA0, zc]), jnp.concatenate([colA1, zc]),
        zc2, zc2, zc2, zc2,
        jnp.concatenate([zc, colD0]), jnp.concatenate([zc, colD1])], axis=1)
    Bt = jnp.stack([
        zc2, zc2,
        jnp.concatenate([zc, colD0]), jnp.concatenate([zc, colD1]),
        jnp.concatenate([colA0, zc]), jnp.concatenate([colA1, zc]),
        zc2, zc2], axis=1)

    ew_r = enrichment_weights.reshape(E // 128, 128)
    es2 = enrichment_scale.reshape(1, 1)

    xs1, xs2, a_cat, fac = pl.pallas_call(
        _prologue_body,
        out_shape=[
            jax.ShapeDtypeStruct((N, HC), f32),
            jax.ShapeDtypeStruct((N, HC), f32),
            jax.ShapeDtypeStruct((N, 8), f32),
            jax.ShapeDtypeStruct((1, 1), f32),
        ],
    )(h_x, t_x, W_cat, Bh, Bt, ew_r, es2)

    sent = jnp.array([[-1e30, -1e30, 0.0, 0.0]], f32)
    padz = jnp.zeros((N_PAD - N - 1, 4), f32)
    a1 = jnp.concatenate([a_cat[:, 0:4], sent, padz], axis=0)
    a2 = jnp.concatenate([a_cat[:, 4:8], sent, padz], axis=0)
    xs_padz = jnp.zeros((N_PAD - N, HC), f32)
    xs1p = jnp.concatenate([xs1, xs_padz], axis=0)
    xs2p = jnp.concatenate([xs2, xs_padz], axis=0)

    padi = jnp.full((E_PAD - E,), N, jnp.int32)
    src_p = jnp.concatenate([edge_index[0], padi]).reshape(NW, KB, EB)
    dst_p = jnp.concatenate([edge_index[1], padi]).reshape(NW, KB, EB)

    den1_p, den2_p, ex1, ex2 = _edge_logits(src_p, dst_p, a1, a2)
    o1 = _messages(xs1p, src_p, dst_p, ex1)
    o2 = _messages(xs2p, dst_p, src_p, ex2)

    den1 = (den1_p[0] + den1_p[1]).reshape(N_PAD, H)[:N]
    den2 = (den2_p[0] + den2_p[1]).reshape(N_PAD, H)[:N]
    d1 = jnp.repeat(den1, C, axis=1) + 1e-16
    d2 = jnp.repeat(den2, C, axis=1) + 1e-16

    t_rep, h_rep = pl.pallas_call(
        _epilogue_body,
        out_shape=[
            jax.ShapeDtypeStruct((N, HC), f32),
            jax.ShapeDtypeStruct((N, HC), f32),
        ],
    )(o1[0, :N], o1[1, :N], o2[0, :N], o2[1, :N], d1, d2,
      bias.reshape(1, HC), fac)
    return (h_rep, t_rep)
